# baseline (speedup 1.0000x reference)
import functools

import jax
import jax.numpy as jnp
from jax import lax
from jax.experimental import pallas as pl
from jax.experimental.pallas import tpu as pltpu
from jax.experimental.pallas import tpu_sc as plsc

N_ACTIONS = 64
L = 16
NW = 32
IDX_CHUNKS = 4
IDX_MINOR = 128
BPW = IDX_CHUNKS * IDX_MINOR
LN2 = 0.6931471805599453
SQRT2 = 1.4142135623730951


def _vlog(x):
    bits = plsc.bitcast(x, jnp.int32)
    e = jnp.right_shift(bits, 23) & 0xFF
    mbits = (bits & 0x007FFFFF) | 0x3F800000
    m = plsc.bitcast(mbits, jnp.float32)
    big = m > SQRT2
    m = jnp.where(big, m * 0.5, m)
    ef = (e - 127).astype(jnp.float32) + jnp.where(big, 1.0, 0.0)
    z = (m - 1.0) / (m + 1.0)
    z2 = z * z
    poly = 1.0 + z2 * (1.0 / 3.0 + z2 * (1.0 / 5.0 + z2 * (1.0 / 7.0 + z2 * (1.0 / 9.0))))
    return ef * LN2 + 2.0 * z * poly


def _make_sc_call():
    mesh = plsc.VectorSubcoreMesh(core_axis_name="c", subcore_axis_name="s")

    @functools.partial(
        pl.kernel,
        mesh=mesh,
        compiler_params=pltpu.CompilerParams(needs_layout_passes=False, use_tc_tiling_on_sc=False),
        out_type=jax.ShapeDtypeStruct((NW, BPW), jnp.float32),
        scratch_types=[
            pltpu.VMEM((IDX_CHUNKS, IDX_MINOR), jnp.int32),
            pltpu.VMEM((BPW, N_ACTIONS), jnp.float32),
            pltpu.VMEM((BPW,), jnp.int32),
            pltpu.VMEM((BPW,), jnp.float32),
            pltpu.SemaphoreType.DMA,
        ],
    )
    def sc_kernel(policy_hbm, feat_hbm, act_hbm, out_hbm,
                  idx_v, rows_v, act_v, out_v, sem):
        wid = lax.axis_index("s") * 2 + lax.axis_index("c")

        pltpu.sync_copy(feat_hbm.at[wid], idx_v)
        pltpu.sync_copy(act_hbm.at[wid], act_v)

        copies = [
            pltpu.async_copy(
                policy_hbm.at[idx_v.at[j]],
                rows_v.at[pl.ds(j * IDX_MINOR, IDX_MINOR)],
                sem,
            )
            for j in range(IDX_CHUNKS)
        ]
        for c in copies:
            c.wait()

        iota = lax.broadcasted_iota(jnp.int32, (L,), 0)

        def group_body(g, carry):
            row_idx = g * L + iota
            m = plsc.load_gather(rows_v, [row_idx, jnp.zeros((L,), jnp.int32)])
            for a in range(1, N_ACTIONS):
                col = jnp.full((L,), a, jnp.int32)
                m = jnp.maximum(m, plsc.load_gather(rows_v, [row_idx, col]))
            s = jnp.zeros((L,), jnp.float32)
            for a in range(N_ACTIONS):
                col = jnp.full((L,), a, jnp.int32)
                v = plsc.load_gather(rows_v, [row_idx, col])
                s = s + jnp.exp(v - m)
            lse = m + _vlog(s)
            acts = plsc.load_gather(act_v, [row_idx])
            taken = plsc.load_gather(rows_v, [row_idx, acts])
            plsc.store_scatter(out_v, [row_idx], taken - lse)
            return carry

        lax.fori_loop(0, BPW // L, group_body, 0)
        pltpu.sync_copy(out_v, out_hbm.at[wid])

    return sc_kernel


_sc_call = _make_sc_call()


def kernel(policy, feat, taken_actions):
    b = feat.shape[0]
    feat3 = feat.reshape(NW, IDX_CHUNKS, IDX_MINOR)
    act2 = taken_actions.reshape(NW, BPW)
    out2 = _sc_call(policy, feat3, act2)
    return out2.reshape(b)


# speedup vs baseline: 1.6293x; 1.6293x over previous
import functools

import jax
import jax.numpy as jnp
from jax import lax
from jax.experimental import pallas as pl
from jax.experimental.pallas import tpu as pltpu
from jax.experimental.pallas import tpu_sc as plsc

N_ACTIONS = 64
L = 16
NW = 32
BPW = 512
NG = BPW // L
ROW_PAD = 128
LN2 = 0.6931471805599453
SQRT2 = 1.4142135623730951


def _vlog(x):
    bits = plsc.bitcast(x, jnp.int32)
    e = jnp.right_shift(bits, 23) & 0xFF
    mbits = (bits & 0x007FFFFF) | 0x3F800000
    m = plsc.bitcast(mbits, jnp.float32)
    big = m > SQRT2
    m = jnp.where(big, m * 0.5, m)
    ef = (e - 127).astype(jnp.float32) + jnp.where(big, 1.0, 0.0)
    z = (m - 1.0) / (m + 1.0)
    z2 = z * z
    poly = 1.0 + z2 * (1.0 / 3.0 + z2 * (1.0 / 5.0 + z2 * (1.0 / 7.0 + z2 * (1.0 / 9.0))))
    return ef * LN2 + 2.0 * z * poly


def _make_sc_call():
    mesh = plsc.VectorSubcoreMesh(core_axis_name="c", subcore_axis_name="s")

    @functools.partial(
        pl.kernel,
        mesh=mesh,
        compiler_params=pltpu.CompilerParams(needs_layout_passes=False),
        out_type=jax.ShapeDtypeStruct((NW * BPW,), jnp.float32),
        scratch_types=[
            pltpu.VMEM((BPW,), jnp.int32),
            pltpu.VMEM((BPW, ROW_PAD), jnp.float32),
            pltpu.VMEM((BPW,), jnp.int32),
            pltpu.VMEM((BPW,), jnp.float32),
            pltpu.SemaphoreType.DMA,
        ],
    )
    def sc_kernel(policy_hbm, feat_hbm, act_hbm, out_hbm,
                  idx_v, rows_v, act_v, out_v, sem):
        wid = lax.axis_index("s") * 2 + lax.axis_index("c")
        base = wid * BPW

        pltpu.sync_copy(feat_hbm.at[pl.ds(base, BPW)], idx_v)
        pltpu.sync_copy(act_hbm.at[pl.ds(base, BPW)], act_v)

        iota = lax.broadcasted_iota(jnp.int32, (L,), 0)

        def gather_group(g, carry):
            copies = []
            ids = idx_v[pl.ds(g * L, L)]
            for j in range(L):
                r = ids[j]
                copies.append(pltpu.async_copy(
                    policy_hbm.at[r],
                    rows_v.at[g * L + j, pl.ds(0, N_ACTIONS)],
                    sem,
                ))
            for c in copies:
                c.wait()
            return carry

        lax.fori_loop(0, NG, gather_group, 0)

        def group_body(g, carry):
            row_idx = g * L + iota
            m = plsc.load_gather(rows_v, [row_idx, jnp.zeros((L,), jnp.int32)])
            for a in range(1, N_ACTIONS):
                col = jnp.full((L,), a, jnp.int32)
                m = jnp.maximum(m, plsc.load_gather(rows_v, [row_idx, col]))
            s = jnp.zeros((L,), jnp.float32)
            for a in range(N_ACTIONS):
                col = jnp.full((L,), a, jnp.int32)
                v = plsc.load_gather(rows_v, [row_idx, col])
                s = s + jnp.exp(v - m)
            lse = m + _vlog(s)
            acts = plsc.load_gather(act_v, [row_idx])
            taken = plsc.load_gather(rows_v, [row_idx, acts])
            plsc.store_scatter(out_v, [row_idx], taken - lse)
            return carry

        lax.fori_loop(0, NG, group_body, 0)
        pltpu.sync_copy(out_v, out_hbm.at[pl.ds(base, BPW)])

    return sc_kernel


_sc_call = _make_sc_call()


def kernel(policy, feat, taken_actions):
    return _sc_call(policy, feat, taken_actions)


# speedup vs baseline: 2.5967x; 1.5937x over previous
import functools

import jax
import jax.numpy as jnp
from jax import lax
from jax.experimental import pallas as pl
from jax.experimental.pallas import tpu as pltpu
from jax.experimental.pallas import tpu_sc as plsc

N_ACTIONS = 64
N_STATES = 1000000
B = 16384
L = 16
NW = 32
W = 256
NWIN = (N_STATES + W - 1) // W
LAST_WIN = NWIN - 1
LAST_WIN_START = LAST_WIN * W
LAST_WIN_LEN = N_STATES - LAST_WIN_START
TPW = (NWIN + NW - 1) // NW
NPAIR = (TPW + 1) // 2
LN2 = 0.6931471805599453
SQRT2 = 1.4142135623730951


def _vlog(x):
    bits = plsc.bitcast(x, jnp.int32)
    e = jnp.right_shift(bits, 23) & 0xFF
    mbits = (bits & 0x007FFFFF) | 0x3F800000
    m = plsc.bitcast(mbits, jnp.float32)
    big = m > SQRT2
    m = jnp.where(big, m * 0.5, m)
    ef = (e - 127).astype(jnp.float32) + jnp.where(big, 1.0, 0.0)
    z = (m - 1.0) / (m + 1.0)
    z2 = z * z
    poly = 1.0 + z2 * (1.0 / 3.0 + z2 * (1.0 / 5.0 + z2 * (1.0 / 7.0 + z2 * (1.0 / 9.0))))
    return ef * LN2 + 2.0 * z * poly


def _make_sc_call():
    mesh = plsc.VectorSubcoreMesh(core_axis_name="c", subcore_axis_name="s")

    @functools.partial(
        pl.kernel,
        mesh=mesh,
        compiler_params=pltpu.CompilerParams(needs_layout_passes=False),
        out_type=jax.ShapeDtypeStruct((B,), jnp.float32),
        scratch_types=[
            pltpu.VMEM((B,), jnp.int32),
            pltpu.VMEM((B,), jnp.int32),
            pltpu.VMEM((B,), jnp.int32),
            pltpu.VMEM((N_ACTIONS, W), jnp.float32),
            pltpu.VMEM((N_ACTIONS, W), jnp.float32),
            pltpu.VMEM((N_ACTIONS, LAST_WIN_LEN), jnp.float32),
            pltpu.VMEM((N_ACTIONS, L), jnp.float32),
            pltpu.VMEM((L,), jnp.int32),
            pltpu.VMEM((L,), jnp.int32),
            pltpu.VMEM((L,), jnp.int32),
            pltpu.VMEM((L,), jnp.int32),
            pltpu.VMEM((L,), jnp.float32),
            pltpu.SemaphoreType.DMA,
            pltpu.SemaphoreType.DMA,
            pltpu.SemaphoreType.DMA,
        ],
    )
    def sc_kernel(pt_hbm, aux_hbm, feat_hbm, act_hbm, out_hbm,
                  fv, av, hits, w0, w1, wtail, tslots, smeta, ch_i, ch_r,
                  f_g, f_v, sem0, sem1, semf):
        wid = lax.axis_index("s") * 2 + lax.axis_index("c")
        iota = lax.broadcasted_iota(jnp.int32, (L,), 0)
        lane0 = iota == 0

        pltpu.sync_copy(feat_hbm, fv)
        pltpu.sync_copy(act_hbm, av)
        pltpu.sync_copy(aux_hbm, wtail)

        def prefilter(k, pos):
            rv = fv[pl.ds(k * L, L)]
            m = (jnp.right_shift(rv, 8) & (NW - 1)) == wid
            mi = m.astype(jnp.int32)
            dest = pos + plsc.cumsum(mi) - 1
            plsc.store_scatter(hits, [dest], k * L + iota, mask=m)
            return pos + plsc.all_reduce_population_count(m)[0]

        nhit = lax.fori_loop(0, B // L, prefilter, 0)
        nchunk = (nhit + (L - 1)) >> 4

        def my_win(t):
            return jnp.minimum(wid + NW * t, LAST_WIN)

        def fire(t, buf, sem):
            widx = my_win(t)
            s = widx * W

            @pl.when(widx != LAST_WIN)
            def _():
                pltpu.async_copy(
                    pt_hbm.at[pl.ds(0, N_ACTIONS), pl.ds(s, W)],
                    buf.at[pl.ds(0, N_ACTIONS), pl.ds(0, W)],
                    sem,
                )

        def drain(t, buf, sem):
            @pl.when(my_win(t) != LAST_WIN)
            def _():
                pltpu.make_async_copy(
                    pt_hbm.at[pl.ds(0, N_ACTIONS), pl.ds(0, W)],
                    buf.at[pl.ds(0, N_ACTIONS), pl.ds(0, W)],
                    sem,
                ).wait()

        def flush(slotcnt):
            nvalid = ((slotcnt - 1) & (L - 1)) + 1
            m = tslots[0]
            for a in range(1, N_ACTIONS):
                m = jnp.maximum(m, tslots[a])
            ssum = jnp.zeros((L,), jnp.float32)
            for a in range(N_ACTIONS):
                ssum = ssum + jnp.exp(tslots[a] - m)
            lse = m + _vlog(ssum)
            valid = iota < nvalid
            gid = jnp.where(valid, smeta[:], 0)
            a16 = plsc.load_gather(av, [gid])
            taken = plsc.load_gather(tslots, [a16, iota])
            val = taken - lse
            gid0 = jnp.full((L,), gid[0], jnp.int32)
            val0 = jnp.full((L,), val[0], jnp.float32)
            f_g[:] = jnp.where(valid, gid, gid0)
            f_v[:] = jnp.where(valid, val, val0)
            pltpu.async_copy(f_v, out_hbm.at[f_g], semf).wait()

        def process(t, buf, slotcnt):
            widx = my_win(t)
            s = widx * W

            def chunk_body(k, sc):
                lanes_ok = (k * L + iota) < nhit
                i16 = jnp.where(lanes_ok, hits[pl.ds(k * L, L)], 0)
                r16 = plsc.load_gather(fv, [i16])
                inwin = (r16 >= s) & (r16 < s + W) & lanes_ok
                cnt = plsc.all_reduce_population_count(inwin)[0]

                @pl.when(cnt > 0)
                def _():
                    dest = plsc.cumsum(inwin.astype(jnp.int32)) - 1
                    plsc.store_scatter(ch_i, [dest], i16, mask=inwin)
                    plsc.store_scatter(ch_r, [dest], r16 - s, mask=inwin)

                def hit_body(h, sc2):
                    hsp = jnp.full((L,), h, jnp.int32)
                    ivec = plsc.load_gather(ch_i, [hsp])
                    rvec = plsc.load_gather(ch_r, [hsp])
                    rloc = rvec[0]
                    slot = sc2 & (L - 1)
                    slotsp = jnp.full((L,), slot, jnp.int32)
                    rlocsp = jnp.full((L,), rloc, jnp.int32)

                    @pl.when(widx != LAST_WIN)
                    def _():
                        for kk in range(N_ACTIONS // L):
                            v = plsc.load_gather(buf, [kk * L + iota, rlocsp])
                            plsc.store_scatter(tslots, [kk * L + iota, slotsp], v)

                    @pl.when(widx == LAST_WIN)
                    def _():
                        for kk in range(N_ACTIONS // L):
                            v = plsc.load_gather(wtail, [kk * L + iota, rlocsp])
                            plsc.store_scatter(tslots, [kk * L + iota, slotsp], v)
                    plsc.store_scatter(smeta, [slotsp], ivec, mask=lane0)
                    sc2 = sc2 + 1

                    @pl.when((sc2 & (L - 1)) == 0)
                    def _():
                        flush(sc2)

                    return sc2

                return lax.fori_loop(0, cnt, hit_body, sc)

            return lax.fori_loop(0, nchunk, chunk_body, slotcnt)

        fire(0, w0, sem0)

        def pair_body(p, slotcnt):
            t0 = 2 * p
            fire(t0 + 1, w1, sem1)
            drain(t0, w0, sem0)
            slotcnt = process(t0, w0, slotcnt)
            fire(t0 + 2, w0, sem0)
            drain(t0 + 1, w1, sem1)
            slotcnt = process(t0 + 1, w1, slotcnt)
            return slotcnt

        slotcnt = lax.fori_loop(0, NPAIR, pair_body, 0)
        drain(2 * NPAIR, w0, sem0)

        @pl.when((slotcnt & (L - 1)) != 0)
        def _():
            flush(slotcnt)

    return sc_kernel


_sc_call = _make_sc_call()


def kernel(policy, feat, taken_actions):
    tail = policy[LAST_WIN_START:].T
    return _sc_call(policy.T, tail, feat, taken_actions)


# speedup vs baseline: 3.6034x; 1.3877x over previous
import functools

import jax
import jax.numpy as jnp
from jax import lax
from jax.experimental import pallas as pl
from jax.experimental.pallas import tpu as pltpu
from jax.experimental.pallas import tpu_sc as plsc

N_ACTIONS = 64
N_STATES = 1000000
B = 16384
L = 16
NW = 32
W = 512
WSHIFT = 9
NWIN = (N_STATES + W - 1) // W
LAST_WIN = NWIN - 1
LAST_WIN_START = LAST_WIN * W
LAST_WIN_LEN = N_STATES - LAST_WIN_START
TPW = (NWIN + NW - 1) // NW
NPAIR = (TPW + 1) // 2
LN2 = 0.6931471805599453
SQRT2 = 1.4142135623730951


def _vlog(x):
    bits = plsc.bitcast(x, jnp.int32)
    e = jnp.right_shift(bits, 23) & 0xFF
    mbits = (bits & 0x007FFFFF) | 0x3F800000
    m = plsc.bitcast(mbits, jnp.float32)
    big = m > SQRT2
    m = jnp.where(big, m * 0.5, m)
    ef = (e - 127).astype(jnp.float32) + jnp.where(big, 1.0, 0.0)
    z = (m - 1.0) / (m + 1.0)
    z2 = z * z
    poly = 1.0 + z2 * (1.0 / 3.0 + z2 * (1.0 / 5.0 + z2 * (1.0 / 7.0 + z2 * (1.0 / 9.0))))
    return ef * LN2 + 2.0 * z * poly


def _make_sc_call():
    mesh = plsc.VectorSubcoreMesh(core_axis_name="c", subcore_axis_name="s")

    @functools.partial(
        pl.kernel,
        mesh=mesh,
        compiler_params=pltpu.CompilerParams(needs_layout_passes=False),
        out_type=jax.ShapeDtypeStruct((B,), jnp.float32),
        scratch_types=[
            pltpu.VMEM((B,), jnp.int32),
            pltpu.VMEM((B,), jnp.int32),
            pltpu.VMEM((N_ACTIONS, W), jnp.float32),
            pltpu.VMEM((N_ACTIONS, W), jnp.float32),
            pltpu.VMEM((N_ACTIONS, LAST_WIN_LEN), jnp.float32),
            pltpu.VMEM((N_ACTIONS, L), jnp.float32),
            pltpu.VMEM((L,), jnp.int32),
            pltpu.VMEM((L,), jnp.int32),
            pltpu.VMEM((L,), jnp.int32),
            pltpu.VMEM((L,), jnp.int32),
            pltpu.VMEM((L,), jnp.float32),
            pltpu.SemaphoreType.DMA,
            pltpu.SemaphoreType.DMA,
            pltpu.SemaphoreType.DMA,
        ],
    )
    def sc_kernel(pt_hbm, aux_hbm, packed_hbm, out_hbm,
                  fv, hits, w0, w1, wtail, tslots, smeta, ch_i, ch_r,
                  f_g, f_v, sem0, sem1, semf):
        wid = lax.axis_index("s") * 2 + lax.axis_index("c")
        iota = lax.broadcasted_iota(jnp.int32, (L,), 0)
        lane0 = iota == 0

        pltpu.sync_copy(packed_hbm, fv)
        pltpu.sync_copy(aux_hbm, wtail)

        def prefilter(k, pos):
            rv = fv[pl.ds(k * L, L)] & 0xFFFFF
            m = (jnp.right_shift(rv, WSHIFT) & (NW - 1)) == wid
            mi = m.astype(jnp.int32)
            dest = pos + plsc.cumsum(mi) - 1
            plsc.store_scatter(hits, [dest], k * L + iota, mask=m)
            return pos + plsc.all_reduce_population_count(m)[0]

        nhit = lax.fori_loop(0, B // L, prefilter, 0)
        nchunk = (nhit + (L - 1)) >> 4

        def my_win(t):
            return jnp.minimum(wid + NW * t, LAST_WIN)

        def fire(t, buf, sem):
            widx = my_win(t)
            s = widx * W

            @pl.when(widx != LAST_WIN)
            def _():
                pltpu.async_copy(
                    pt_hbm.at[pl.ds(0, N_ACTIONS), pl.ds(s, W)],
                    buf.at[pl.ds(0, N_ACTIONS), pl.ds(0, W)],
                    sem,
                )

        def drain(t, buf, sem):
            @pl.when(my_win(t) != LAST_WIN)
            def _():
                pltpu.make_async_copy(
                    pt_hbm.at[pl.ds(0, N_ACTIONS), pl.ds(0, W)],
                    buf.at[pl.ds(0, N_ACTIONS), pl.ds(0, W)],
                    sem,
                ).wait()

        def flush(slotcnt):
            nvalid = ((slotcnt - 1) & (L - 1)) + 1
            m = tslots[0]
            for a in range(1, N_ACTIONS):
                m = jnp.maximum(m, tslots[a])
            ssum = jnp.zeros((L,), jnp.float32)
            for a in range(N_ACTIONS):
                ssum = ssum + jnp.exp(tslots[a] - m)
            lse = m + _vlog(ssum)
            valid = iota < nvalid
            gid = jnp.where(valid, smeta[:], 0)
            a16 = jnp.right_shift(plsc.load_gather(fv, [gid]), 20)
            taken = plsc.load_gather(tslots, [a16, iota])
            val = taken - lse
            gid0 = jnp.full((L,), gid[0], jnp.int32)
            val0 = jnp.full((L,), val[0], jnp.float32)
            f_g[:] = jnp.where(valid, gid, gid0)
            f_v[:] = jnp.where(valid, val, val0)
            pltpu.async_copy(f_v, out_hbm.at[f_g], semf).wait()

        def process(t, buf, slotcnt):
            widx = my_win(t)
            s = widx * W

            def chunk_body(k, sc):
                lanes_ok = (k * L + iota) < nhit
                i16 = jnp.where(lanes_ok, hits[pl.ds(k * L, L)], 0)
                r16 = plsc.load_gather(fv, [i16]) & 0xFFFFF
                inwin = (r16 >= s) & (r16 < s + W) & lanes_ok
                cnt = plsc.all_reduce_population_count(inwin)[0]

                @pl.when(cnt > 0)
                def _():
                    dest = plsc.cumsum(inwin.astype(jnp.int32)) - 1
                    plsc.store_scatter(ch_i, [dest], i16, mask=inwin)
                    plsc.store_scatter(ch_r, [dest], r16 - s, mask=inwin)

                def hit_body(h, sc2):
                    hsp = jnp.full((L,), h, jnp.int32)
                    ivec = plsc.load_gather(ch_i, [hsp])
                    rvec = plsc.load_gather(ch_r, [hsp])
                    rloc = rvec[0]
                    slot = sc2 & (L - 1)
                    slotsp = jnp.full((L,), slot, jnp.int32)
                    rlocsp = jnp.full((L,), rloc, jnp.int32)

                    @pl.when(widx != LAST_WIN)
                    def _():
                        for kk in range(N_ACTIONS // L):
                            v = plsc.load_gather(buf, [kk * L + iota, rlocsp])
                            plsc.store_scatter(tslots, [kk * L + iota, slotsp], v)

                    @pl.when(widx == LAST_WIN)
                    def _():
                        for kk in range(N_ACTIONS // L):
                            v = plsc.load_gather(wtail, [kk * L + iota, rlocsp])
                            plsc.store_scatter(tslots, [kk * L + iota, slotsp], v)
                    plsc.store_scatter(smeta, [slotsp], ivec, mask=lane0)
                    sc2 = sc2 + 1

                    @pl.when((sc2 & (L - 1)) == 0)
                    def _():
                        flush(sc2)

                    return sc2

                return lax.fori_loop(0, cnt, hit_body, sc)

            return lax.fori_loop(0, nchunk, chunk_body, slotcnt)

        fire(0, w0, sem0)

        def pair_body(p, slotcnt):
            t0 = 2 * p
            fire(t0 + 1, w1, sem1)
            drain(t0, w0, sem0)
            slotcnt = process(t0, w0, slotcnt)
            fire(t0 + 2, w0, sem0)
            drain(t0 + 1, w1, sem1)
            slotcnt = process(t0 + 1, w1, slotcnt)
            return slotcnt

        slotcnt = lax.fori_loop(0, NPAIR, pair_body, 0)
        drain(2 * NPAIR, w0, sem0)

        @pl.when((slotcnt & (L - 1)) != 0)
        def _():
            flush(slotcnt)

    return sc_kernel


_sc_call = _make_sc_call()


def kernel(policy, feat, taken_actions):
    tail = policy[LAST_WIN_START:].T
    packed = jnp.bitwise_or(jnp.left_shift(taken_actions, 20), feat)
    return _sc_call(policy.T, tail, packed)


# speedup vs baseline: 3.6660x; 1.0174x over previous
import functools

import jax
import jax.numpy as jnp
from jax import lax
from jax.experimental import pallas as pl
from jax.experimental.pallas import tpu as pltpu
from jax.experimental.pallas import tpu_sc as plsc

N_ACTIONS = 64
N_STATES = 1000000
B = 16384
L = 16
NW = 32
W = 512
WSHIFT = 9
NWIN = (N_STATES + W - 1) // W
LAST_WIN = NWIN - 1
LAST_WIN_START = LAST_WIN * W
LAST_WIN_LEN = N_STATES - LAST_WIN_START
TPW = (NWIN + NW - 1) // NW
NPAIR = (TPW + 1) // 2
LN2 = 0.6931471805599453
SQRT2 = 1.4142135623730951


def _vlog(x):
    bits = plsc.bitcast(x, jnp.int32)
    e = jnp.right_shift(bits, 23) & 0xFF
    mbits = (bits & 0x007FFFFF) | 0x3F800000
    m = plsc.bitcast(mbits, jnp.float32)
    big = m > SQRT2
    m = jnp.where(big, m * 0.5, m)
    ef = (e - 127).astype(jnp.float32) + jnp.where(big, 1.0, 0.0)
    z = (m - 1.0) / (m + 1.0)
    z2 = z * z
    poly = 1.0 + z2 * (1.0 / 3.0 + z2 * (1.0 / 5.0 + z2 * (1.0 / 7.0 + z2 * (1.0 / 9.0))))
    return ef * LN2 + 2.0 * z * poly


def _make_sc_call():
    mesh = plsc.VectorSubcoreMesh(core_axis_name="c", subcore_axis_name="s")

    @functools.partial(
        pl.kernel,
        mesh=mesh,
        compiler_params=pltpu.CompilerParams(needs_layout_passes=False),
        out_type=jax.ShapeDtypeStruct((B,), jnp.float32),
        scratch_types=[
            pltpu.VMEM((B,), jnp.int32),
            pltpu.VMEM((B,), jnp.int32),
            pltpu.VMEM((N_ACTIONS, W), jnp.float32),
            pltpu.VMEM((N_ACTIONS, W), jnp.float32),
            pltpu.VMEM((N_ACTIONS, LAST_WIN_LEN), jnp.float32),
            pltpu.VMEM((N_ACTIONS, L), jnp.float32),
            pltpu.VMEM((L,), jnp.int32),
            pltpu.VMEM((L,), jnp.int32),
            pltpu.VMEM((L,), jnp.int32),
            pltpu.VMEM((L,), jnp.int32),
            pltpu.VMEM((L,), jnp.float32),
            pltpu.SemaphoreType.DMA,
            pltpu.SemaphoreType.DMA,
            pltpu.SemaphoreType.DMA,
        ],
    )
    def sc_kernel(pt_hbm, aux_hbm, packed_hbm, out_hbm,
                  fv, hits, w0, w1, wtail, tslots, smeta, ch_i, ch_r,
                  f_g, f_v, sem0, sem1, semf):
        wid = lax.axis_index("s") * 2 + lax.axis_index("c")
        iota = lax.broadcasted_iota(jnp.int32, (L,), 0)
        lane0 = iota == 0

        pltpu.sync_copy(packed_hbm, fv)
        pltpu.sync_copy(aux_hbm, wtail)

        def prefilter(k, pos):
            rv = fv[pl.ds(k * L, L)] & 0xFFFFF
            m = (jnp.right_shift(rv, WSHIFT) & (NW - 1)) == wid
            mi = m.astype(jnp.int32)
            dest = pos + plsc.cumsum(mi) - 1
            plsc.store_scatter(hits, [dest], k * L + iota, mask=m)
            return pos + plsc.all_reduce_population_count(m)[0]

        def my_win(t):
            return jnp.minimum(wid + NW * t, LAST_WIN)

        def fire(t, buf, sem):
            widx = my_win(t)
            s = widx * W

            @pl.when(widx != LAST_WIN)
            def _():
                pltpu.async_copy(
                    pt_hbm.at[pl.ds(0, N_ACTIONS), pl.ds(s, W)],
                    buf.at[pl.ds(0, N_ACTIONS), pl.ds(0, W)],
                    sem,
                )

        def drain(t, buf, sem):
            @pl.when(my_win(t) != LAST_WIN)
            def _():
                pltpu.make_async_copy(
                    pt_hbm.at[pl.ds(0, N_ACTIONS), pl.ds(0, W)],
                    buf.at[pl.ds(0, N_ACTIONS), pl.ds(0, W)],
                    sem,
                ).wait()

        fire(0, w0, sem0)
        fire(1, w1, sem1)
        nhit = lax.fori_loop(0, B // L, prefilter, 0)
        nchunk = (nhit + (L - 1)) >> 4

        def flush(slotcnt):
            nvalid = ((slotcnt - 1) & (L - 1)) + 1
            m = tslots[0]
            for a in range(1, N_ACTIONS):
                m = jnp.maximum(m, tslots[a])
            ssum = jnp.zeros((L,), jnp.float32)
            for a in range(N_ACTIONS):
                ssum = ssum + jnp.exp(tslots[a] - m)
            lse = m + _vlog(ssum)
            valid = iota < nvalid
            gid = jnp.where(valid, smeta[:], 0)
            a16 = jnp.right_shift(plsc.load_gather(fv, [gid]), 20)
            taken = plsc.load_gather(tslots, [a16, iota])
            val = taken - lse
            gid0 = jnp.full((L,), gid[0], jnp.int32)
            val0 = jnp.full((L,), val[0], jnp.float32)
            f_g[:] = jnp.where(valid, gid, gid0)
            f_v[:] = jnp.where(valid, val, val0)
            pltpu.async_copy(f_v, out_hbm.at[f_g], semf).wait()

        def process(t, buf, slotcnt):
            widx = my_win(t)
            s = widx * W

            def chunk_body(k, sc):
                lanes_ok = (k * L + iota) < nhit
                i16 = jnp.where(lanes_ok, hits[pl.ds(k * L, L)], 0)
                r16 = plsc.load_gather(fv, [i16]) & 0xFFFFF
                inwin = (r16 >= s) & (r16 < s + W) & lanes_ok
                cnt = plsc.all_reduce_population_count(inwin)[0]

                @pl.when(cnt > 0)
                def _():
                    dest = plsc.cumsum(inwin.astype(jnp.int32)) - 1
                    plsc.store_scatter(ch_i, [dest], i16, mask=inwin)
                    plsc.store_scatter(ch_r, [dest], r16 - s, mask=inwin)

                def hit_body(h, sc2):
                    hsp = jnp.full((L,), h, jnp.int32)
                    ivec = plsc.load_gather(ch_i, [hsp])
                    rvec = plsc.load_gather(ch_r, [hsp])
                    rloc = rvec[0]
                    slot = sc2 & (L - 1)
                    slotsp = jnp.full((L,), slot, jnp.int32)
                    rlocsp = jnp.full((L,), rloc, jnp.int32)

                    @pl.when(widx != LAST_WIN)
                    def _():
                        for kk in range(N_ACTIONS // L):
                            v = plsc.load_gather(buf, [kk * L + iota, rlocsp])
                            plsc.store_scatter(tslots, [kk * L + iota, slotsp], v)

                    @pl.when(widx == LAST_WIN)
                    def _():
                        for kk in range(N_ACTIONS // L):
                            v = plsc.load_gather(wtail, [kk * L + iota, rlocsp])
                            plsc.store_scatter(tslots, [kk * L + iota, slotsp], v)
                    plsc.store_scatter(smeta, [slotsp], ivec, mask=lane0)
                    sc2 = sc2 + 1

                    @pl.when((sc2 & (L - 1)) == 0)
                    def _():
                        flush(sc2)

                    return sc2

                return lax.fori_loop(0, cnt, hit_body, sc)

            return lax.fori_loop(0, nchunk, chunk_body, slotcnt)


        def pair_body(p, slotcnt):
            t0 = 2 * p
            drain(t0, w0, sem0)
            slotcnt = process(t0, w0, slotcnt)
            fire(t0 + 2, w0, sem0)
            drain(t0 + 1, w1, sem1)
            slotcnt = process(t0 + 1, w1, slotcnt)
            fire(t0 + 3, w1, sem1)
            return slotcnt

        slotcnt = lax.fori_loop(0, NPAIR, pair_body, 0)

        @pl.when((slotcnt & (L - 1)) != 0)
        def _():
            flush(slotcnt)

    return sc_kernel


_sc_call = _make_sc_call()


def kernel(policy, feat, taken_actions):
    tail = policy[LAST_WIN_START:].T
    packed = jnp.bitwise_or(jnp.left_shift(taken_actions, 20), feat)
    return _sc_call(policy.T, tail, packed)
